# baseline (speedup 1.0000x reference)
import dataclasses
import functools

import jax
import jax.numpy as jnp
from jax import lax
from jax.experimental import pallas as pl
from jax.experimental.pallas import tpu as pltpu
from jax.experimental.pallas import tpu_sc as plsc

N = 10000
E = 320000
IN_DIM = 128
HID = 128
HEADS = 8
DH = HID // HEADS
OUT_DIM = 16

NPAD = 10240
RB = 1024


def _mm_kernel(x_ref, w_ref, b_ref, s_ref, o_ref):
    acc = jnp.dot(x_ref[...], w_ref[...], preferred_element_type=jnp.float32)
    o_ref[...] = (acc + b_ref[...]) * s_ref[...]


def _matmul_scaled(x, w, b, s):
    m = x.shape[0]
    k = x.shape[1]
    n = w.shape[1]
    grid = (m // RB,)
    return pl.pallas_call(
        _mm_kernel,
        grid=grid,
        in_specs=[
            pl.BlockSpec((RB, k), lambda i: (i, 0)),
            pl.BlockSpec((k, n), lambda i: (0, 0)),
            pl.BlockSpec((1, n), lambda i: (0, 0)),
            pl.BlockSpec((RB, 1), lambda i: (i, 0)),
        ],
        out_specs=pl.BlockSpec((RB, n), lambda i: (i, 0)),
        out_shape=jax.ShapeDtypeStruct((m, n), jnp.float32),
    )(x, w, b.reshape(1, n), s.reshape(m, 1))


_SC_MESH = plsc.VectorSubcoreMesh(core_axis_name="c", subcore_axis_name="s")
_SC_CP = pltpu.CompilerParams()
if "needs_layout_passes" in pltpu.CompilerParams.__dataclass_fields__:
    _SC_CP = dataclasses.replace(_SC_CP, needs_layout_passes=False)
NCORE = 2
NSUB = 16
EPAD = 327680
CHUNK = 1024
JROWS = CHUNK // 128
VROWS = 256
ROWS_PER_TILE = NPAD // NSUB


def _sc_gather_scatter_add(table_pad, src2d, dst2d, linear=False):
    rows_per_core = src2d.shape[0] // NCORE
    chunks = EPAD // (NCORE * CHUNK)
    iters = chunks // NSUB

    @functools.partial(
        pl.kernel,
        mesh=_SC_MESH,
        out_type=jax.ShapeDtypeStruct((NCORE, NPAD, HID), jnp.float32),
        scratch_types=[
            pltpu.VMEM((JROWS, 128), jnp.int32),
            pltpu.VMEM((JROWS, 128), jnp.int32),
            pltpu.VMEM((VROWS, HID), jnp.float32),
            pltpu.VMEM_SHARED((NPAD, HID), jnp.float32),
        ],
    )
    def k(table_hbm, src_hbm, dst_hbm, out_hbm, sidx, didx, vals, acc):
        c = lax.axis_index("c")
        s = lax.axis_index("s")

        @pl.loop(0, VROWS)
        def _(r):
            @pl.loop(0, HID // 16)
            def _(kk):
                vals[r, pl.ds(kk * 16, 16)] = jnp.zeros((16,), jnp.float32)

        off = 0
        while off < ROWS_PER_TILE:
            ln = min(ROWS_PER_TILE - off, VROWS)
            pltpu.sync_copy(vals.at[pl.ds(0, ln)],
                            acc.at[pl.ds(s * ROWS_PER_TILE + off, ln)])
            off += ln
        plsc.subcore_barrier()

        @pl.loop(0, iters)
        def _(it):
            i = it * NSUB + s
            row_off = c * rows_per_core + i * JROWS
            if not linear:
                pltpu.sync_copy(src_hbm.at[pl.ds(row_off, JROWS)], sidx)
            pltpu.sync_copy(dst_hbm.at[pl.ds(row_off, JROWS)], didx)
            jq = VROWS // 128
            for q in range(JROWS // jq):
                if linear:
                    pltpu.sync_copy(
                        table_hbm.at[pl.ds((row_off + q * jq) * 128, VROWS)],
                        vals)
                else:
                    for j in range(jq):
                        pltpu.sync_copy(
                            table_hbm.at[sidx.at[q * jq + j]],
                            vals.at[pl.ds(j * 128, 128)])
                for j in range(jq):
                    pltpu.sync_copy(
                        vals.at[pl.ds(j * 128, 128)],
                        acc.at[didx.at[q * jq + j]], add=True)

        plsc.subcore_barrier()
        pltpu.sync_copy(acc.at[pl.ds(s * ROWS_PER_TILE, ROWS_PER_TILE)],
                        out_hbm.at[c].at[pl.ds(s * ROWS_PER_TILE,
                                               ROWS_PER_TILE)])

    return k(table_pad, src2d, dst2d)


GROWS = 512


def _sc_gather(table_pad, idx2d):
    rows_per_core = idx2d.shape[0] // NCORE
    chunks = EPAD // (NCORE * CHUNK)
    iters = chunks // NSUB

    @functools.partial(
        pl.kernel,
        mesh=_SC_MESH,
        out_type=jax.ShapeDtypeStruct((EPAD, HID), jnp.float32),
        scratch_types=[
            pltpu.VMEM((JROWS, 128), jnp.int32),
            pltpu.VMEM((JROWS, 128), jnp.int32),
            pltpu.VMEM((256, HID), jnp.float32),
            pltpu.VMEM((256, HID), jnp.float32),
            pltpu.SemaphoreType.DMA,
            pltpu.SemaphoreType.DMA,
            pltpu.SemaphoreType.DMA,
        ],
    )
    def k(table_hbm, idx_hbm, out_hbm, idx0, idx1, v0, v1,
          sem_i, sem_g, sem_o):
        c = lax.axis_index("c")
        s = lax.axis_index("s")

        def row_off(it):
            return c * rows_per_core + (it * NSUB + s) * JROWS

        pltpu.sync_copy(idx_hbm.at[pl.ds(row_off(0), JROWS)], idx0)
        idx_bufs = [idx0, idx1]
        val_bufs = [v0, v1]
        out_pending = [None, None]
        idx_pending = [None, None]
        for it in range(iters):
            ib = idx_bufs[it % 2]
            if idx_pending[it % 2] is not None:
                idx_pending[it % 2].wait()
                idx_pending[it % 2] = None
            if it + 1 < iters:
                idx_pending[(it + 1) % 2] = pltpu.async_copy(
                    idx_hbm.at[pl.ds(row_off(it + 1), JROWS)],
                    idx_bufs[(it + 1) % 2], sem_i)
            for q in range(JROWS // 2):
                vb = val_bufs[q % 2]
                if out_pending[q % 2] is not None:
                    out_pending[q % 2].wait()
                g0 = pltpu.async_copy(table_hbm.at[ib.at[2 * q]],
                                      vb.at[pl.ds(0, 128)], sem_g)
                g1 = pltpu.async_copy(table_hbm.at[ib.at[2 * q + 1]],
                                      vb.at[pl.ds(128, 128)], sem_g)
                g0.wait()
                g1.wait()
                out_pending[q % 2] = pltpu.async_copy(
                    vb,
                    out_hbm.at[pl.ds(row_off(it) * 128 + q * 256, 256)],
                    sem_o)
        for p in out_pending:
            if p is not None:
                p.wait()

    return k(table_pad, idx2d)


WCAP = 4096
DCH = 512
NGRP = (NPAD // (NCORE * NSUB)) // 16
_NEG = -jnp.inf


def _sc_softmax(logits_flat, rp0, rp1):
    nodes_per_tile = NPAD // (NCORE * NSUB)

    @functools.partial(
        pl.kernel,
        mesh=_SC_MESH,
        compiler_params=_SC_CP,
        out_type=jax.ShapeDtypeStruct((EPAD * 8,), jnp.float32),
        scratch_types=[
            pltpu.VMEM((nodes_per_tile,), jnp.int32),
            pltpu.VMEM((nodes_per_tile,), jnp.int32),
            pltpu.VMEM((WCAP * 8,), jnp.float32),
            pltpu.VMEM((WCAP * 8,), jnp.float32),
            pltpu.VMEM((8, 16), jnp.float32),
            pltpu.VMEM((8, 16), jnp.float32),
        ],
    )
    def k(lg_flat, rp0_hbm, rp1_hbm, out_flat,
          rp0v, rp1v, lbuf, abuf, mbuf, sbuf):
        c = lax.axis_index("c")
        s = lax.axis_index("s")
        wid = c * NSUB + s
        nlo = wid * nodes_per_tile

        pltpu.sync_copy(rp0_hbm.at[pl.ds(nlo, nodes_per_tile)], rp0v)
        pltpu.sync_copy(rp1_hbm.at[pl.ds(nlo, nodes_per_tile)], rp1v)

        def dma_window_in(wb, wlen):
            nch = (wlen + DCH - 1) // DCH

            def body(ci, _):
                pltpu.sync_copy(
                    lg_flat.at[pl.ds((wb + ci * DCH) * 8, DCH * 8)],
                    lbuf.at[pl.ds(ci * DCH * 8, DCH * 8)])
                return 0

            lax.fori_loop(0, nch, body, 0)

        def dma_window_out(wb, wlen):
            full = wlen // DCH

            def body(ci, _):
                pltpu.sync_copy(
                    abuf.at[pl.ds(ci * DCH * 8, DCH * 8)],
                    out_flat.at[pl.ds((wb + ci * DCH) * 8, DCH * 8)])
                return 0

            lax.fori_loop(0, full, body, 0)
            rem = wlen - full * DCH
            base = full * DCH

            def body64(ci, _):
                pltpu.sync_copy(
                    abuf.at[pl.ds((base + ci * 64) * 8, 64 * 8)],
                    out_flat.at[pl.ds((wb + base + ci * 64) * 8, 64 * 8)])
                return 0

            lax.fori_loop(0, rem // 64, body64, 0)
            rem8 = rem - (rem // 64) * 64
            base8 = base + (rem // 64) * 64

            def body8(ci, _):
                pltpu.sync_copy(
                    abuf.at[pl.ds((base8 + ci * 8) * 8, 8 * 8)],
                    out_flat.at[pl.ds((wb + base8 + ci * 8) * 8, 8 * 8)])
                return 0

            lax.fori_loop(0, rem8 // 8, body8, 0)
            rem1 = rem8 - (rem8 // 8) * 8
            base1 = base8 + (rem8 // 8) * 8

            def body1(ci, _):
                pltpu.sync_copy(
                    abuf.at[pl.ds((base1 + ci) * 8, 8)],
                    out_flat.at[pl.ds((wb + base1 + ci) * 8, 8)])
                return 0

            lax.fori_loop(0, rem1, body1, 0)

        @pl.loop(0, NGRP)
        def _(g):
            nb = g * 16
            rp0_vec = rp0v[pl.ds(nb, 16)]
            rp1_vec = rp1v[pl.ds(nb, 16)]
            deg_vec = rp1_vec - rp0_vec
            ge_start = jnp.min(rp0_vec)
            cnt = jnp.max(rp1_vec) - ge_start
            nwin = (cnt + WCAP - 1) // WCAP

            for h in range(8):
                mbuf[h, pl.ds(0, 16)] = jnp.full((16,), _NEG, jnp.float32)
                sbuf[h, pl.ds(0, 16)] = jnp.zeros((16,), jnp.float32)

            def win_bounds(w):
                wb = ge_start + w * WCAP
                wlen = jnp.minimum(cnt - w * WCAP, WCAP)
                base_vec = rp0_vec - wb
                jlo = jnp.maximum(-base_vec, 0)
                jhi = jnp.minimum(deg_vec, wlen - base_vec)
                jmin = jnp.min(jlo)
                jmax = jnp.max(jhi)
                return wb, wlen, base_vec, jlo, jhi, jmin, jmax

            def idx_of(base_vec, j, h):
                idx = (base_vec + j) * 8 + h
                return jnp.clip(idx, 0, WCAP * 8 - 1)

            def w1(w, _):
                wb, wlen, base_vec, jlo, jhi, jmin, jmax = win_bounds(w)
                dma_window_in(wb, wlen)
                for h in range(8):
                    def jb(carry):
                        j, m = carry
                        mask = (j >= jlo) & (j < jhi)
                        val = plsc.load_gather(
                            lbuf, [idx_of(base_vec, j, h)], mask=mask)
                        m = jnp.maximum(
                            m, jnp.where(mask, val,
                                         jnp.full((16,), _NEG, jnp.float32)))
                        return j + 1, m

                    _, m = lax.while_loop(
                        lambda cr: cr[0] < jmax, jb,
                        (jmin, mbuf[h, pl.ds(0, 16)]))
                    mbuf[h, pl.ds(0, 16)] = m
                return 0

            lax.fori_loop(0, nwin, w1, 0)

            def w2(w, _):
                wb, wlen, base_vec, jlo, jhi, jmin, jmax = win_bounds(w)
                dma_window_in(wb, wlen)
                for h in range(8):
                    m = mbuf[h, pl.ds(0, 16)]

                    def jb(carry):
                        j, acc = carry
                        mask = (j >= jlo) & (j < jhi)
                        val = plsc.load_gather(
                            lbuf, [idx_of(base_vec, j, h)], mask=mask)
                        ex = jnp.exp(val - m)
                        acc = acc + jnp.where(mask, ex,
                                              jnp.zeros((16,), jnp.float32))
                        return j + 1, acc

                    _, acc = lax.while_loop(
                        lambda cr: cr[0] < jmax, jb,
                        (jmin, sbuf[h, pl.ds(0, 16)]))
                    sbuf[h, pl.ds(0, 16)] = acc
                return 0

            lax.fori_loop(0, nwin, w2, 0)

            for h in range(8):
                d = sbuf[h, pl.ds(0, 16)]
                sbuf[h, pl.ds(0, 16)] = 1.0 / jnp.maximum(d, 1e-9)

            def w3(w, _):
                wb, wlen, base_vec, jlo, jhi, jmin, jmax = win_bounds(w)
                dma_window_in(wb, wlen)
                for h in range(8):
                    m = mbuf[h, pl.ds(0, 16)]
                    invd = sbuf[h, pl.ds(0, 16)]

                    def jb(carry):
                        j = carry
                        mask = (j >= jlo) & (j < jhi)
                        idx = idx_of(base_vec, j, h)
                        val = plsc.load_gather(lbuf, [idx], mask=mask)
                        a = jnp.exp(val - m) * invd
                        plsc.store_scatter(abuf, [idx], a, mask=mask)
                        return j + 1

                    lax.while_loop(lambda j: j < jmax, jb, jmin)
                dma_window_out(wb, wlen)
                return 0

            lax.fori_loop(0, nwin, w3, 0)

    return k(logits_flat, rp0, rp1)


RBE = 2048


def _logits_kernel(hs_ref, hd_ref, a_ref, o_ref):
    z = hs_ref[...] + hd_ref[...]
    t = jnp.maximum(z, 0.2 * z) * a_ref[...]
    col = lax.broadcasted_iota(jnp.int32, (HID, HID), 0) // DH
    row = lax.broadcasted_iota(jnp.int32, (HID, HID), 1)
    g = (col == row).astype(jnp.float32)
    lg = jnp.dot(t, g, preferred_element_type=jnp.float32)
    o_ref[...] = lg[:, :8]


def _tc_logits(hs_e, hd_e, attn_flat):
    grid = (EPAD // RBE,)
    return pl.pallas_call(
        _logits_kernel,
        grid=grid,
        in_specs=[
            pl.BlockSpec((RBE, HID), lambda i: (i, 0)),
            pl.BlockSpec((RBE, HID), lambda i: (i, 0)),
            pl.BlockSpec((1, HID), lambda i: (0, 0)),
        ],
        out_specs=pl.BlockSpec((RBE, 8), lambda i: (i, 0)),
        out_shape=jax.ShapeDtypeStruct((EPAD, 8), jnp.float32),
    )(hs_e, hd_e, attn_flat.reshape(1, HID))


def _weighted_kernel(a_ref, hs_ref, o_ref):
    i = pl.program_id(0)
    r = lax.broadcasted_iota(jnp.int32, (8, HID), 0)
    cc = lax.broadcasted_iota(jnp.int32, (8, HID), 1) // DH
    rmat = (r == cc).astype(jnp.float32)
    a128 = jnp.dot(a_ref[...], rmat, preferred_element_type=jnp.float32)
    erow = i * RBE + lax.broadcasted_iota(jnp.int32, (RBE, 1), 0)
    o_ref[...] = jnp.where(erow < E, a128 * hs_ref[...], 0.0)


def _tc_weighted(alpha, hs_e):
    grid = (EPAD // RBE,)
    return pl.pallas_call(
        _weighted_kernel,
        grid=grid,
        in_specs=[
            pl.BlockSpec((RBE, 8), lambda i: (i, 0)),
            pl.BlockSpec((RBE, HID), lambda i: (i, 0)),
        ],
        out_specs=pl.BlockSpec((RBE, HID), lambda i: (i, 0)),
        out_shape=jax.ShapeDtypeStruct((EPAD, HID), jnp.float32),
    )(alpha, hs_e)


def _residual_kernel(p_ref, h_ref, o_ref):
    o_ref[...] = jnp.maximum(p_ref[0] + p_ref[1] + h_ref[...], 0.0)


def _combine_residual_relu(partials, h):
    grid = (NPAD // RB,)
    return pl.pallas_call(
        _residual_kernel,
        grid=grid,
        in_specs=[
            pl.BlockSpec((NCORE, RB, HID), lambda i: (0, i, 0)),
            pl.BlockSpec((RB, HID), lambda i: (i, 0)),
        ],
        out_specs=pl.BlockSpec((RB, HID), lambda i: (i, 0)),
        out_shape=jax.ShapeDtypeStruct((NPAD, HID), jnp.float32),
    )(partials, h)


def _combine_kernel(p_ref, s_ref, b_ref, o_ref):
    o_ref[...] = jnp.maximum(
        (p_ref[0] + p_ref[1]) * s_ref[...] + b_ref[...], 0.0)


def _combine_scale_bias_relu(partials, s, b):
    grid = (NPAD // RB,)
    return pl.pallas_call(
        _combine_kernel,
        grid=grid,
        in_specs=[
            pl.BlockSpec((NCORE, RB, HID), lambda i: (0, i, 0)),
            pl.BlockSpec((RB, 1), lambda i: (i, 0)),
            pl.BlockSpec((1, HID), lambda i: (0, 0)),
        ],
        out_specs=pl.BlockSpec((RB, HID), lambda i: (i, 0)),
        out_shape=jax.ShapeDtypeStruct((NPAD, HID), jnp.float32),
    )(partials, s.reshape(NPAD, 1), b.reshape(1, HID))


def _final_kernel(h_ref, w1_ref, b1_ref, w2_ref, b2_ref, w3_ref, b3_ref,
                  o_ref, mx_ref):
    i = pl.program_id(0)

    @pl.when(i == 0)
    def _():
        mx_ref[...] = jnp.full_like(mx_ref, -jnp.inf)

    mx_ref[...] = jnp.maximum(mx_ref[...], jnp.max(h_ref[...], axis=0,
                                                   keepdims=True))

    @pl.when(i == pl.num_programs(0) - 1)
    def _():
        hg = mx_ref[...]
        h1 = jnp.maximum(jnp.dot(hg, w1_ref[...],
                                 preferred_element_type=jnp.float32)
                         + b1_ref[...], 0.0)
        h2 = jnp.maximum(jnp.dot(h1, w2_ref[...],
                                 preferred_element_type=jnp.float32)
                         + b2_ref[...], 0.0)
        o_ref[...] = jnp.dot(h2, w3_ref[...],
                             preferred_element_type=jnp.float32) + b3_ref[...]


def _final_pool_mlp(h, w1, b1, w2, b2, w3, b3):
    m = h.shape[0]
    grid = (m // RB,)
    return pl.pallas_call(
        _final_kernel,
        grid=grid,
        in_specs=[
            pl.BlockSpec((RB, HID), lambda i: (i, 0)),
            pl.BlockSpec((HID, HID), lambda i: (0, 0)),
            pl.BlockSpec((1, HID), lambda i: (0, 0)),
            pl.BlockSpec((HID, HID // 2), lambda i: (0, 0)),
            pl.BlockSpec((1, HID // 2), lambda i: (0, 0)),
            pl.BlockSpec((HID // 2, OUT_DIM), lambda i: (0, 0)),
            pl.BlockSpec((1, OUT_DIM), lambda i: (0, 0)),
        ],
        out_specs=pl.BlockSpec((1, OUT_DIM), lambda i: (0, 0)),
        out_shape=jax.ShapeDtypeStruct((1, OUT_DIM), jnp.float32),
        scratch_shapes=[pltpu.VMEM((1, HID), jnp.float32)],
    )(h, w1, b1.reshape(1, -1), w2, b2.reshape(1, -1), w3, b3.reshape(1, -1))


def kernel(x, edge_index, gc1_w, gc1_b, gc2_w, gc2_b, gat_wsrc, gat_bsrc,
           gat_wdst, gat_bdst, gat_attn, cls_w1, cls_b1, cls_w2, cls_b2,
           cls_w3, cls_b3):
    n = x.shape[0]
    src = edge_index[0]
    dst = edge_index[1]

    order = jnp.argsort(dst)
    dst_s = dst[order]
    src_s = src[order]
    rp0 = jnp.searchsorted(dst_s, jnp.arange(NPAD, dtype=jnp.int32)
                           ).astype(jnp.int32)
    rp1 = jnp.searchsorted(dst_s, jnp.arange(1, NPAD + 1, dtype=jnp.int32)
                           ).astype(jnp.int32)
    src_sorted = jnp.sort(src)
    op0 = jnp.searchsorted(src_sorted, jnp.arange(n, dtype=jnp.int32))
    op1 = jnp.searchsorted(src_sorted, jnp.arange(1, n + 1, dtype=jnp.int32))
    deg_out = jnp.maximum((op1 - op0).astype(jnp.float32), 1.0)
    deg_in = jnp.maximum((rp1[:n] - rp0[:n]).astype(jnp.float32), 1.0)
    do_isqrt = jax.lax.rsqrt(deg_out)
    di_isqrt = jax.lax.rsqrt(deg_in)

    pad_rows = NPAD - n
    xp = jnp.pad(x, ((0, pad_rows), (0, 0)))
    do_p = jnp.pad(do_isqrt, (0, pad_rows))
    di_p = jnp.pad(di_isqrt, (0, pad_rows))
    onesn = jnp.ones((NPAD,), jnp.float32)

    srcs2d = jnp.concatenate(
        [src_s, jnp.full((EPAD - E,), NPAD, jnp.int32)]).reshape(
            EPAD // 128, 128)
    dsts2d = jnp.concatenate(
        [dst_s, jnp.zeros((EPAD - E,), jnp.int32)]).reshape(EPAD // 128, 128)
    ztail = jnp.zeros((8, HID), jnp.float32)

    def gcn(hp, w, b):
        hm = _matmul_scaled(hp, w, jnp.zeros_like(b), do_p)
        partials = _sc_gather_scatter_add(
            jnp.concatenate([hm, ztail], axis=0), srcs2d, dsts2d)
        return _combine_scale_bias_relu(partials, di_p, b)

    h = gcn(xp, gc1_w, gc1_b)
    h = gcn(h, gc2_w, gc2_b)

    for i in range(3):
        hs = _matmul_scaled(h, gat_wsrc[i], gat_bsrc[i], onesn)
        hd = _matmul_scaled(h, gat_wdst[i], gat_bdst[i], onesn)
        hs_e = _sc_gather(jnp.concatenate([hs, ztail], axis=0), srcs2d)
        hd_e = _sc_gather(jnp.concatenate([hd, ztail], axis=0), dsts2d)
        logits = _tc_logits(hs_e, hd_e, gat_attn[i].reshape(HID))
        alpha = _sc_softmax(logits.reshape(EPAD * 8), rp0, rp1)
        weighted = _tc_weighted(alpha.reshape(EPAD, 8), hs_e)
        partials = _sc_gather_scatter_add(weighted, dsts2d, dsts2d,
                                          linear=True)
        h = _combine_residual_relu(partials, h)

    hmasked = jnp.where(
        (jnp.arange(NPAD) < n)[:, None], h, -jnp.inf)
    return _final_pool_mlp(hmasked, cls_w1, cls_b1, cls_w2, cls_b2,
                           cls_w3, cls_b3)


# speedup vs baseline: 1.1128x; 1.1128x over previous
import dataclasses
import functools

import jax
import jax.numpy as jnp
from jax import lax
from jax.experimental import pallas as pl
from jax.experimental.pallas import tpu as pltpu
from jax.experimental.pallas import tpu_sc as plsc

N = 10000
E = 320000
IN_DIM = 128
HID = 128
HEADS = 8
DH = HID // HEADS
OUT_DIM = 16

NPAD = 10240
RB = 1024


def _mm_kernel(x_ref, w_ref, b_ref, s_ref, o_ref):
    acc = jnp.dot(x_ref[...], w_ref[...], preferred_element_type=jnp.float32)
    o_ref[...] = (acc + b_ref[...]) * s_ref[...]


def _matmul_scaled(x, w, b, s):
    m = x.shape[0]
    k = x.shape[1]
    n = w.shape[1]
    grid = (m // RB,)
    return pl.pallas_call(
        _mm_kernel,
        grid=grid,
        in_specs=[
            pl.BlockSpec((RB, k), lambda i: (i, 0)),
            pl.BlockSpec((k, n), lambda i: (0, 0)),
            pl.BlockSpec((1, n), lambda i: (0, 0)),
            pl.BlockSpec((RB, 1), lambda i: (i, 0)),
        ],
        out_specs=pl.BlockSpec((RB, n), lambda i: (i, 0)),
        out_shape=jax.ShapeDtypeStruct((m, n), jnp.float32),
    )(x, w, b.reshape(1, n), s.reshape(m, 1))


_SC_MESH = plsc.VectorSubcoreMesh(core_axis_name="c", subcore_axis_name="s")
_SC_CP = pltpu.CompilerParams()
if "needs_layout_passes" in pltpu.CompilerParams.__dataclass_fields__:
    _SC_CP = dataclasses.replace(_SC_CP, needs_layout_passes=False)
NCORE = 2
NSUB = 16
EPAD = 327680
CHUNK = 1024
JROWS = CHUNK // 128
VROWS = 256
ROWS_PER_TILE = NPAD // NSUB


def _sc_gather_scatter_add(table_pad, src2d, dst2d, linear=False):
    rows_per_core = src2d.shape[0] // NCORE
    chunks = EPAD // (NCORE * CHUNK)
    iters = chunks // NSUB

    @functools.partial(
        pl.kernel,
        mesh=_SC_MESH,
        out_type=jax.ShapeDtypeStruct((NCORE, NPAD, HID), jnp.float32),
        scratch_types=[
            pltpu.VMEM((JROWS, 128), jnp.int32),
            pltpu.VMEM((JROWS, 128), jnp.int32),
            pltpu.VMEM((VROWS, HID), jnp.float32),
            pltpu.VMEM_SHARED((NPAD, HID), jnp.float32),
        ],
    )
    def k(table_hbm, src_hbm, dst_hbm, out_hbm, sidx, didx, vals, acc):
        c = lax.axis_index("c")
        s = lax.axis_index("s")

        @pl.loop(0, VROWS)
        def _(r):
            @pl.loop(0, HID // 16)
            def _(kk):
                vals[r, pl.ds(kk * 16, 16)] = jnp.zeros((16,), jnp.float32)

        off = 0
        while off < ROWS_PER_TILE:
            ln = min(ROWS_PER_TILE - off, VROWS)
            pltpu.sync_copy(vals.at[pl.ds(0, ln)],
                            acc.at[pl.ds(s * ROWS_PER_TILE + off, ln)])
            off += ln
        plsc.subcore_barrier()

        @pl.loop(0, iters)
        def _(it):
            i = it * NSUB + s
            row_off = c * rows_per_core + i * JROWS
            if not linear:
                pltpu.sync_copy(src_hbm.at[pl.ds(row_off, JROWS)], sidx)
            pltpu.sync_copy(dst_hbm.at[pl.ds(row_off, JROWS)], didx)
            jq = VROWS // 128
            for q in range(JROWS // jq):
                if linear:
                    pltpu.sync_copy(
                        table_hbm.at[pl.ds((row_off + q * jq) * 128, VROWS)],
                        vals)
                else:
                    for j in range(jq):
                        pltpu.sync_copy(
                            table_hbm.at[sidx.at[q * jq + j]],
                            vals.at[pl.ds(j * 128, 128)])
                for j in range(jq):
                    pltpu.sync_copy(
                        vals.at[pl.ds(j * 128, 128)],
                        acc.at[didx.at[q * jq + j]], add=True)

        plsc.subcore_barrier()
        pltpu.sync_copy(acc.at[pl.ds(s * ROWS_PER_TILE, ROWS_PER_TILE)],
                        out_hbm.at[c].at[pl.ds(s * ROWS_PER_TILE,
                                               ROWS_PER_TILE)])

    return k(table_pad, src2d, dst2d)


GROWS = 512


def _sc_gather(table_pad, idx2d):
    e_total = idx2d.shape[0] * 128
    rows_per_core = idx2d.shape[0] // NCORE
    iters = e_total // (NCORE * CHUNK * NSUB)

    @functools.partial(
        pl.kernel,
        mesh=_SC_MESH,
        out_type=jax.ShapeDtypeStruct((e_total, HID), jnp.float32),
        scratch_types=[
            pltpu.VMEM((JROWS, 128), jnp.int32),
            pltpu.VMEM((JROWS, 128), jnp.int32),
            pltpu.VMEM((256, HID), jnp.float32),
            pltpu.VMEM((256, HID), jnp.float32),
            pltpu.SemaphoreType.DMA,
            pltpu.SemaphoreType.DMA,
            pltpu.SemaphoreType.DMA,
        ],
    )
    def k(table_hbm, idx_hbm, out_hbm, idx0, idx1, v0, v1,
          sem_i, sem_g, sem_o):
        c = lax.axis_index("c")
        s = lax.axis_index("s")

        def row_off(it):
            return c * rows_per_core + (it * NSUB + s) * JROWS

        pltpu.sync_copy(idx_hbm.at[pl.ds(row_off(0), JROWS)], idx0)
        idx_bufs = [idx0, idx1]
        val_bufs = [v0, v1]
        out_pending = [None, None]
        idx_pending = [None, None]
        for it in range(iters):
            ib = idx_bufs[it % 2]
            if idx_pending[it % 2] is not None:
                idx_pending[it % 2].wait()
                idx_pending[it % 2] = None
            if it + 1 < iters:
                idx_pending[(it + 1) % 2] = pltpu.async_copy(
                    idx_hbm.at[pl.ds(row_off(it + 1), JROWS)],
                    idx_bufs[(it + 1) % 2], sem_i)
            for q in range(JROWS // 2):
                vb = val_bufs[q % 2]
                if out_pending[q % 2] is not None:
                    out_pending[q % 2].wait()
                g0 = pltpu.async_copy(table_hbm.at[ib.at[2 * q]],
                                      vb.at[pl.ds(0, 128)], sem_g)
                g1 = pltpu.async_copy(table_hbm.at[ib.at[2 * q + 1]],
                                      vb.at[pl.ds(128, 128)], sem_g)
                g0.wait()
                g1.wait()
                out_pending[q % 2] = pltpu.async_copy(
                    vb,
                    out_hbm.at[pl.ds(row_off(it) * 128 + q * 256, 256)],
                    sem_o)
        for p in out_pending:
            if p is not None:
                p.wait()

    return k(table_pad, idx2d)


WCAP = 4096
DCH = 512
NGRP = (NPAD // (NCORE * NSUB)) // 16
_NEG = -jnp.inf


def _sc_softmax(logits_flat, rp0, rp1):
    nodes_per_tile = NPAD // (NCORE * NSUB)

    @functools.partial(
        pl.kernel,
        mesh=_SC_MESH,
        compiler_params=_SC_CP,
        out_type=jax.ShapeDtypeStruct((EPAD * 8,), jnp.float32),
        scratch_types=[
            pltpu.VMEM((nodes_per_tile,), jnp.int32),
            pltpu.VMEM((nodes_per_tile,), jnp.int32),
            pltpu.VMEM((WCAP * 8,), jnp.float32),
            pltpu.VMEM((WCAP * 8,), jnp.float32),
            pltpu.VMEM((8, 16), jnp.float32),
            pltpu.VMEM((8, 16), jnp.float32),
        ],
    )
    def k(lg_flat, rp0_hbm, rp1_hbm, out_flat,
          rp0v, rp1v, lbuf, abuf, mbuf, sbuf):
        c = lax.axis_index("c")
        s = lax.axis_index("s")
        wid = c * NSUB + s
        nlo = wid * nodes_per_tile

        pltpu.sync_copy(rp0_hbm.at[pl.ds(nlo, nodes_per_tile)], rp0v)
        pltpu.sync_copy(rp1_hbm.at[pl.ds(nlo, nodes_per_tile)], rp1v)

        def dma_window_in(wb, wlen):
            nch = (wlen + DCH - 1) // DCH

            def body(ci, _):
                pltpu.sync_copy(
                    lg_flat.at[pl.ds((wb + ci * DCH) * 8, DCH * 8)],
                    lbuf.at[pl.ds(ci * DCH * 8, DCH * 8)])
                return 0

            lax.fori_loop(0, nch, body, 0)

        def dma_window_out(wb, wlen):
            full = wlen // DCH

            def body(ci, _):
                pltpu.sync_copy(
                    abuf.at[pl.ds(ci * DCH * 8, DCH * 8)],
                    out_flat.at[pl.ds((wb + ci * DCH) * 8, DCH * 8)])
                return 0

            lax.fori_loop(0, full, body, 0)
            rem = wlen - full * DCH
            base = full * DCH

            def body64(ci, _):
                pltpu.sync_copy(
                    abuf.at[pl.ds((base + ci * 64) * 8, 64 * 8)],
                    out_flat.at[pl.ds((wb + base + ci * 64) * 8, 64 * 8)])
                return 0

            lax.fori_loop(0, rem // 64, body64, 0)
            rem8 = rem - (rem // 64) * 64
            base8 = base + (rem // 64) * 64

            def body8(ci, _):
                pltpu.sync_copy(
                    abuf.at[pl.ds((base8 + ci * 8) * 8, 8 * 8)],
                    out_flat.at[pl.ds((wb + base8 + ci * 8) * 8, 8 * 8)])
                return 0

            lax.fori_loop(0, rem8 // 8, body8, 0)
            rem1 = rem8 - (rem8 // 8) * 8
            base1 = base8 + (rem8 // 8) * 8

            def body1(ci, _):
                pltpu.sync_copy(
                    abuf.at[pl.ds((base1 + ci) * 8, 8)],
                    out_flat.at[pl.ds((wb + base1 + ci) * 8, 8)])
                return 0

            lax.fori_loop(0, rem1, body1, 0)

        @pl.loop(0, NGRP)
        def _(g):
            nb = g * 16
            rp0_vec = rp0v[pl.ds(nb, 16)]
            rp1_vec = rp1v[pl.ds(nb, 16)]
            deg_vec = rp1_vec - rp0_vec
            ge_start = jnp.min(rp0_vec)
            cnt = jnp.max(rp1_vec) - ge_start
            nwin = (cnt + WCAP - 1) // WCAP

            for h in range(8):
                mbuf[h, pl.ds(0, 16)] = jnp.full((16,), _NEG, jnp.float32)
                sbuf[h, pl.ds(0, 16)] = jnp.zeros((16,), jnp.float32)

            def win_bounds(w):
                wb = ge_start + w * WCAP
                wlen = jnp.minimum(cnt - w * WCAP, WCAP)
                base_vec = rp0_vec - wb
                jlo = jnp.maximum(-base_vec, 0)
                jhi = jnp.minimum(deg_vec, wlen - base_vec)
                jmin = jnp.min(jlo)
                jmax = jnp.max(jhi)
                return wb, wlen, base_vec, jlo, jhi, jmin, jmax

            def idx_of(base_vec, j, h):
                idx = (base_vec + j) * 8 + h
                return jnp.clip(idx, 0, WCAP * 8 - 1)

            def w1(w, _):
                wb, wlen, base_vec, jlo, jhi, jmin, jmax = win_bounds(w)
                dma_window_in(wb, wlen)
                for h in range(8):
                    def jb(carry):
                        j, m = carry
                        mask = (j >= jlo) & (j < jhi)
                        val = plsc.load_gather(
                            lbuf, [idx_of(base_vec, j, h)], mask=mask)
                        m = jnp.maximum(
                            m, jnp.where(mask, val,
                                         jnp.full((16,), _NEG, jnp.float32)))
                        return j + 1, m

                    _, m = lax.while_loop(
                        lambda cr: cr[0] < jmax, jb,
                        (jmin, mbuf[h, pl.ds(0, 16)]))
                    mbuf[h, pl.ds(0, 16)] = m
                return 0

            lax.fori_loop(0, nwin, w1, 0)

            def w2(w, _):
                wb, wlen, base_vec, jlo, jhi, jmin, jmax = win_bounds(w)
                dma_window_in(wb, wlen)
                for h in range(8):
                    m = mbuf[h, pl.ds(0, 16)]

                    def jb(carry):
                        j, acc = carry
                        mask = (j >= jlo) & (j < jhi)
                        val = plsc.load_gather(
                            lbuf, [idx_of(base_vec, j, h)], mask=mask)
                        ex = jnp.exp(val - m)
                        acc = acc + jnp.where(mask, ex,
                                              jnp.zeros((16,), jnp.float32))
                        return j + 1, acc

                    _, acc = lax.while_loop(
                        lambda cr: cr[0] < jmax, jb,
                        (jmin, sbuf[h, pl.ds(0, 16)]))
                    sbuf[h, pl.ds(0, 16)] = acc
                return 0

            lax.fori_loop(0, nwin, w2, 0)

            for h in range(8):
                d = sbuf[h, pl.ds(0, 16)]
                sbuf[h, pl.ds(0, 16)] = 1.0 / jnp.maximum(d, 1e-9)

            def w3(w, _):
                wb, wlen, base_vec, jlo, jhi, jmin, jmax = win_bounds(w)
                dma_window_in(wb, wlen)
                for h in range(8):
                    m = mbuf[h, pl.ds(0, 16)]
                    invd = sbuf[h, pl.ds(0, 16)]

                    def jb(carry):
                        j = carry
                        mask = (j >= jlo) & (j < jhi)
                        idx = idx_of(base_vec, j, h)
                        val = plsc.load_gather(lbuf, [idx], mask=mask)
                        a = jnp.exp(val - m) * invd
                        plsc.store_scatter(abuf, [idx], a, mask=mask)
                        return j + 1

                    lax.while_loop(lambda j: j < jmax, jb, jmin)
                dma_window_out(wb, wlen)
                return 0

            lax.fori_loop(0, nwin, w3, 0)

    return k(logits_flat, rp0, rp1)


RBE = 2048


def _logits_kernel(hs_ref, hd_ref, a_ref, o_ref):
    z = hs_ref[...] + hd_ref[...]
    t = jnp.maximum(z, 0.2 * z) * a_ref[...]
    col = lax.broadcasted_iota(jnp.int32, (HID, HID), 0) // DH
    row = lax.broadcasted_iota(jnp.int32, (HID, HID), 1)
    g = (col == row).astype(jnp.float32)
    lg = jnp.dot(t, g, preferred_element_type=jnp.float32)
    o_ref[...] = lg[:, :8]


def _tc_logits(both_e, attn_flat):
    grid = (EPAD // RBE,)
    return pl.pallas_call(
        _logits_kernel,
        grid=grid,
        in_specs=[
            pl.BlockSpec((RBE, HID), lambda i: (i, 0)),
            pl.BlockSpec((RBE, HID), lambda i: (EPAD // RBE + i, 0)),
            pl.BlockSpec((1, HID), lambda i: (0, 0)),
        ],
        out_specs=pl.BlockSpec((RBE, 8), lambda i: (i, 0)),
        out_shape=jax.ShapeDtypeStruct((EPAD, 8), jnp.float32),
    )(both_e, both_e, attn_flat.reshape(1, HID))


def _weighted_kernel(a_ref, hs_ref, o_ref):
    i = pl.program_id(0)
    r = lax.broadcasted_iota(jnp.int32, (8, HID), 0)
    cc = lax.broadcasted_iota(jnp.int32, (8, HID), 1) // DH
    rmat = (r == cc).astype(jnp.float32)
    a128 = jnp.dot(a_ref[...], rmat, preferred_element_type=jnp.float32)
    erow = i * RBE + lax.broadcasted_iota(jnp.int32, (RBE, 1), 0)
    o_ref[...] = jnp.where(erow < E, a128 * hs_ref[...], 0.0)


def _tc_weighted(alpha, hs_e):
    grid = (EPAD // RBE,)
    return pl.pallas_call(
        _weighted_kernel,
        grid=grid,
        in_specs=[
            pl.BlockSpec((RBE, 8), lambda i: (i, 0)),
            pl.BlockSpec((RBE, HID), lambda i: (i, 0)),
        ],
        out_specs=pl.BlockSpec((RBE, HID), lambda i: (i, 0)),
        out_shape=jax.ShapeDtypeStruct((EPAD, HID), jnp.float32),
    )(alpha, hs_e)


def _residual_kernel(p_ref, h_ref, o_ref):
    o_ref[...] = jnp.maximum(p_ref[0] + p_ref[1] + h_ref[...], 0.0)


def _combine_residual_relu(partials, h):
    grid = (NPAD // RB,)
    return pl.pallas_call(
        _residual_kernel,
        grid=grid,
        in_specs=[
            pl.BlockSpec((NCORE, RB, HID), lambda i: (0, i, 0)),
            pl.BlockSpec((RB, HID), lambda i: (i, 0)),
        ],
        out_specs=pl.BlockSpec((RB, HID), lambda i: (i, 0)),
        out_shape=jax.ShapeDtypeStruct((NPAD, HID), jnp.float32),
    )(partials, h)


def _combine_kernel(p_ref, s_ref, b_ref, o_ref):
    o_ref[...] = jnp.maximum(
        (p_ref[0] + p_ref[1]) * s_ref[...] + b_ref[...], 0.0)


def _combine_scale_bias_relu(partials, s, b):
    grid = (NPAD // RB,)
    return pl.pallas_call(
        _combine_kernel,
        grid=grid,
        in_specs=[
            pl.BlockSpec((NCORE, RB, HID), lambda i: (0, i, 0)),
            pl.BlockSpec((RB, 1), lambda i: (i, 0)),
            pl.BlockSpec((1, HID), lambda i: (0, 0)),
        ],
        out_specs=pl.BlockSpec((RB, HID), lambda i: (i, 0)),
        out_shape=jax.ShapeDtypeStruct((NPAD, HID), jnp.float32),
    )(partials, s.reshape(NPAD, 1), b.reshape(1, HID))


def _final_kernel(h_ref, w1_ref, b1_ref, w2_ref, b2_ref, w3_ref, b3_ref,
                  o_ref, mx_ref):
    i = pl.program_id(0)

    @pl.when(i == 0)
    def _():
        mx_ref[...] = jnp.full_like(mx_ref, -jnp.inf)

    mx_ref[...] = jnp.maximum(mx_ref[...], jnp.max(h_ref[...], axis=0,
                                                   keepdims=True))

    @pl.when(i == pl.num_programs(0) - 1)
    def _():
        hg = mx_ref[...]
        h1 = jnp.maximum(jnp.dot(hg, w1_ref[...],
                                 preferred_element_type=jnp.float32)
                         + b1_ref[...], 0.0)
        h2 = jnp.maximum(jnp.dot(h1, w2_ref[...],
                                 preferred_element_type=jnp.float32)
                         + b2_ref[...], 0.0)
        o_ref[...] = jnp.dot(h2, w3_ref[...],
                             preferred_element_type=jnp.float32) + b3_ref[...]


def _final_pool_mlp(h, w1, b1, w2, b2, w3, b3):
    m = h.shape[0]
    grid = (m // RB,)
    return pl.pallas_call(
        _final_kernel,
        grid=grid,
        in_specs=[
            pl.BlockSpec((RB, HID), lambda i: (i, 0)),
            pl.BlockSpec((HID, HID), lambda i: (0, 0)),
            pl.BlockSpec((1, HID), lambda i: (0, 0)),
            pl.BlockSpec((HID, HID // 2), lambda i: (0, 0)),
            pl.BlockSpec((1, HID // 2), lambda i: (0, 0)),
            pl.BlockSpec((HID // 2, OUT_DIM), lambda i: (0, 0)),
            pl.BlockSpec((1, OUT_DIM), lambda i: (0, 0)),
        ],
        out_specs=pl.BlockSpec((1, OUT_DIM), lambda i: (0, 0)),
        out_shape=jax.ShapeDtypeStruct((1, OUT_DIM), jnp.float32),
        scratch_shapes=[pltpu.VMEM((1, HID), jnp.float32)],
    )(h, w1, b1.reshape(1, -1), w2, b2.reshape(1, -1), w3, b3.reshape(1, -1))


def kernel(x, edge_index, gc1_w, gc1_b, gc2_w, gc2_b, gat_wsrc, gat_bsrc,
           gat_wdst, gat_bdst, gat_attn, cls_w1, cls_b1, cls_w2, cls_b2,
           cls_w3, cls_b3):
    n = x.shape[0]
    src = edge_index[0]
    dst = edge_index[1]

    order = jnp.argsort(dst)
    dst_s = dst[order]
    src_s = src[order]
    rp0 = jnp.searchsorted(dst_s, jnp.arange(NPAD, dtype=jnp.int32)
                           ).astype(jnp.int32)
    rp1 = jnp.searchsorted(dst_s, jnp.arange(1, NPAD + 1, dtype=jnp.int32)
                           ).astype(jnp.int32)
    ones = jnp.ones(src.shape, x.dtype)
    deg_out = jnp.maximum(jnp.zeros((n,), x.dtype).at[src].add(ones), 1.0)
    deg_in = jnp.maximum((rp1[:n] - rp0[:n]).astype(jnp.float32), 1.0)
    do_isqrt = jax.lax.rsqrt(deg_out)
    di_isqrt = jax.lax.rsqrt(deg_in)

    pad_rows = NPAD - n
    xp = jnp.pad(x, ((0, pad_rows), (0, 0)))
    do_p = jnp.pad(do_isqrt, (0, pad_rows))
    di_p = jnp.pad(di_isqrt, (0, pad_rows))
    onesn = jnp.ones((NPAD,), jnp.float32)

    srcs2d = jnp.concatenate(
        [src_s, jnp.full((EPAD - E,), NPAD, jnp.int32)]).reshape(
            EPAD // 128, 128)
    dsts2d = jnp.concatenate(
        [dst_s, jnp.zeros((EPAD - E,), jnp.int32)]).reshape(EPAD // 128, 128)
    ztail = jnp.zeros((8, HID), jnp.float32)

    def gcn(hp, w, b):
        hm = _matmul_scaled(hp, w, jnp.zeros_like(b), do_p)
        partials = _sc_gather_scatter_add(
            jnp.concatenate([hm, ztail], axis=0), srcs2d, dsts2d)
        return _combine_scale_bias_relu(partials, di_p, b)

    h = gcn(xp, gc1_w, gc1_b)
    h = gcn(h, gc2_w, gc2_b)

    idx_both = jnp.concatenate(
        [srcs2d, dsts2d + (NPAD + 8)], axis=0)

    for i in range(3):
        hs = _matmul_scaled(h, gat_wsrc[i], gat_bsrc[i], onesn)
        hd = _matmul_scaled(h, gat_wdst[i], gat_bdst[i], onesn)
        both_e = _sc_gather(
            jnp.concatenate([hs, ztail, hd, ztail], axis=0), idx_both)
        logits = _tc_logits(both_e, gat_attn[i].reshape(HID))
        alpha = _sc_softmax(logits.reshape(EPAD * 8), rp0, rp1)
        weighted = _tc_weighted(alpha.reshape(EPAD, 8), both_e)
        partials = _sc_gather_scatter_add(weighted, dsts2d, dsts2d,
                                          linear=True)
        h = _combine_residual_relu(partials, h)

    hmasked = jnp.where(
        (jnp.arange(NPAD) < n)[:, None], h, -jnp.inf)
    return _final_pool_mlp(hmasked, cls_w1, cls_b1, cls_w2, cls_b2,
                           cls_w3, cls_b3)
